# baseline (speedup 1.0000x reference)
import functools

import jax
import jax.numpy as jnp
from jax import lax
from jax.experimental import pallas as pl
from jax.experimental.pallas import tpu as pltpu
from jax.experimental.pallas import tpu_sc as plsc

NC = 2
NS = 16
NW = NC * NS
CHUNK = 128
NBUF = 4


def _unpack_rows(pk_ref, flat_ref, nrow):
  @pl.loop(0, nrow)
  def _(i):
    for p in range(8):
      flat_ref[8 * i + p] = pk_ref[i, pl.ds(p * 16, 16)]


def _pack_rows(flat_ref, pk_ref, nrow):
  @pl.loop(0, nrow)
  def _(i):
    for p in range(8):
      pk_ref[i, pl.ds(p * 16, 16)] = flat_ref[8 * i + p]


def _make_sc_agg(n_pad, k, feat):
  stripe = n_pad // NS
  stripe_pk = stripe // 8
  mesh = plsc.VectorSubcoreMesh(core_axis_name="c", subcore_axis_name="s")
  assert k % NBUF == 0 and stripe % 8 == 0

  @functools.partial(
      pl.kernel,
      out_type=(jax.ShapeDtypeStruct((NC, n_pad // 8, 128), jnp.float32),
                jax.ShapeDtypeStruct((NC, n_pad, feat), jnp.float32)),
      mesh=mesh,
      compiler_params=pltpu.CompilerParams(use_tc_tiling_on_sc=False),
      scratch_types=[
          pltpu.VMEM((k, CHUNK), jnp.int32),
          pltpu.VMEM((k, CHUNK), jnp.int32),
          pltpu.VMEM((NBUF, CHUNK, feat), jnp.float32),
          pltpu.VMEM((stripe, feat), jnp.float32),
          pltpu.VMEM((stripe_pk, 128), jnp.float32),
          pltpu.VMEM_SHARED((n_pad, feat), jnp.float32),
          pltpu.SemaphoreType.DMA((NBUF,)),
          pltpu.SemaphoreType.DMA((NBUF,)),
          pltpu.SemaphoreType.DMA,
      ],
  )
  def agg(g_hbm, src_hbm, dst_hbm, zeros_hbm, out_hbm, gtab_hbm,
          src_v, dst_v, rows_v, flat_v, pk_v, acc, gsem, ssem, sem):
    c = lax.axis_index("c")
    s = lax.axis_index("s")
    w = c * NS + s
    pltpu.sync_copy(src_hbm.at[pl.ds(w * k, k)], src_v)
    pltpu.sync_copy(dst_hbm.at[pl.ds(w * k, k)], dst_v)
    pltpu.sync_copy(g_hbm.at[pl.ds(s * stripe_pk, stripe_pk)], pk_v)
    _unpack_rows(pk_v, flat_v, stripe_pk)
    pltpu.async_copy(flat_v, gtab_hbm.at[c, pl.ds(s * stripe, stripe)], sem)
    pltpu.sync_copy(zeros_hbm.at[pl.ds(s * stripe, stripe)],
                    acc.at[pl.ds(s * stripe, stripe)])
    pltpu.make_async_copy(
        flat_v, gtab_hbm.at[c, pl.ds(s * stripe, stripe)], sem).wait()
    plsc.subcore_barrier()

    gtab = gtab_hbm.at[c]
    for b in range(NBUF):
      pltpu.async_copy(gtab.at[src_v.at[b]], rows_v.at[b], gsem.at[b])

    @pl.loop(0, k, step=NBUF)
    def _(j):
      for b in range(NBUF):
        cj = j + b
        pltpu.make_async_copy(gtab.at[src_v.at[cj]], rows_v.at[b],
                              gsem.at[b]).wait()
        pltpu.async_copy(rows_v.at[b], acc.at[dst_v.at[cj]], ssem.at[b],
                         add=True)
        bq = (b - 1) % NBUF
        pv = cj - 1

        @pl.when(jnp.logical_and(pv >= 0, pv + NBUF < k))
        def _():
          pltpu.make_async_copy(rows_v.at[bq], acc.at[dst_v.at[0]],
                                ssem.at[bq]).wait()
          pltpu.async_copy(gtab.at[src_v.at[pv + NBUF]], rows_v.at[bq],
                           gsem.at[bq])

    for b in range(NBUF):
      pltpu.make_async_copy(rows_v.at[b], acc.at[dst_v.at[0]],
                            ssem.at[b]).wait()

    plsc.subcore_barrier()
    pltpu.sync_copy(acc.at[pl.ds(s * stripe, stripe)], flat_v)
    _pack_rows(flat_v, pk_v, stripe_pk)
    pltpu.sync_copy(pk_v, out_hbm.at[c, pl.ds(s * stripe_pk, stripe_pk)])

  return agg


def _make_sc_deg(n_pad, k, feat):
  stripe = n_pad // NS
  stripe_pk = stripe // 8
  mesh = plsc.VectorSubcoreMesh(core_axis_name="c", subcore_axis_name="s")

  @functools.partial(
      pl.kernel,
      out_type=jax.ShapeDtypeStruct((NC, n_pad // 8, 128), jnp.float32),
      mesh=mesh,
      compiler_params=pltpu.CompilerParams(use_tc_tiling_on_sc=False),
      scratch_types=[
          pltpu.VMEM((k, CHUNK), jnp.int32),
          pltpu.VMEM((CHUNK, feat), jnp.float32),
          pltpu.VMEM((stripe, feat), jnp.float32),
          pltpu.VMEM((stripe_pk, 128), jnp.float32),
          pltpu.VMEM_SHARED((n_pad, feat), jnp.float32),
          pltpu.SemaphoreType.DMA((NBUF,)),
      ],
  )
  def deg(ones_hbm, dst_hbm, zeros_hbm, out_hbm,
          dst_v, ones_v, flat_v, pk_v, acc, ssem):
    c = lax.axis_index("c")
    s = lax.axis_index("s")
    w = c * NS + s
    pltpu.sync_copy(dst_hbm.at[pl.ds(w * k, k)], dst_v)
    pltpu.sync_copy(ones_hbm, ones_v)
    pltpu.sync_copy(zeros_hbm.at[pl.ds(s * stripe, stripe)],
                    acc.at[pl.ds(s * stripe, stripe)])
    plsc.subcore_barrier()

    for b in range(NBUF):
      pltpu.async_copy(ones_v, acc.at[dst_v.at[b]], ssem.at[b], add=True)

    @pl.loop(NBUF, k, step=NBUF)
    def _(j):
      for b in range(NBUF):
        pltpu.make_async_copy(ones_v, acc.at[dst_v.at[0]], ssem.at[b]).wait()
        pltpu.async_copy(ones_v, acc.at[dst_v.at[j + b]], ssem.at[b],
                         add=True)

    for b in range(NBUF):
      pltpu.make_async_copy(ones_v, acc.at[dst_v.at[0]], ssem.at[b]).wait()

    plsc.subcore_barrier()
    pltpu.sync_copy(acc.at[pl.ds(s * stripe, stripe)], flat_v)
    _pack_rows(flat_v, pk_v, stripe_pk)
    pltpu.sync_copy(pk_v, out_hbm.at[c, pl.ds(s * stripe_pk, stripe_pk)])

  return deg


def kernel(x, edge_index, W1, b1, W2, b2):
  n, d = x.shape
  h = W1.shape[1]
  ncls = W2.shape[1]
  e = edge_index.shape[1]
  feat = 16

  k = -(-e // (NW * CHUNK))
  k += k % 2
  e_pad = NW * k * CHUNK
  stripe = -(-n // NS)
  stripe += (-stripe) % 8
  n_pad = NS * stripe
  npk = n_pad // 8
  n_trash = n_pad - n
  pad_cnt = e_pad - e
  ew = e // 128

  def prep_body(ei_ref, src_ref, dst_ref):
    src_ref[:ew] = ei_ref[0:1, :].reshape(ew, 128)
    dst_ref[:ew] = ei_ref[1:2, :].reshape(ew, 128)
    if pad_cnt:
      i = (lax.broadcasted_iota(jnp.int32, (pad_cnt // 128, 128), 0) * 128
           + lax.broadcasted_iota(jnp.int32, (pad_cnt // 128, 128), 1))
      src_ref[ew:] = (i * 97) % n
      dst_ref[ew:] = n + i % n_trash

  src, dst = pl.pallas_call(
      prep_body,
      out_shape=(jax.ShapeDtypeStruct((NW * k, CHUNK), jnp.int32),
                 jax.ShapeDtypeStruct((NW * k, CHUNK), jnp.int32)),
  )(edge_index)

  zeros = jnp.zeros((n_pad, feat), jnp.float32)
  ones_rows = jnp.ones((CHUNK, feat), jnp.float32)
  W2p = jnp.pad(W2, ((0, 0), (0, feat - ncls)))
  W2bd = jnp.kron(jnp.eye(8, dtype=jnp.float32), W2p)
  vrow = (jnp.arange(feat) < ncls).astype(jnp.float32)[:, None]
  Msum = jnp.kron(jnp.eye(8, dtype=jnp.float32),
                  vrow * jnp.ones((1, feat), jnp.float32))
  b1t = jnp.tile(b1, 8).reshape(1, 128)
  b2t = jnp.tile(jnp.pad(b2, (0, feat - ncls)), 8).reshape(1, 128)

  sc_agg = _make_sc_agg(n_pad, k, feat)
  sc_deg = _make_sc_deg(n_pad, k, feat)

  def mm1_body(x_ref, w_ref, o_ref):
    zpad = jnp.zeros((npk - n // 8, feat), jnp.float32)
    for p in range(8):
      hp = jnp.dot(x_ref[p::8, :], w_ref[...],
                   preferred_element_type=jnp.float32)
      o_ref[:, p * feat:(p + 1) * feat] = jnp.concatenate([hp, zpad], axis=0)

  h1 = pl.pallas_call(
      mm1_body,
      out_shape=jax.ShapeDtypeStruct((npk, 128), jnp.float32),
  )(x, W1)

  deg_p = sc_deg(ones_rows, dst, zeros)

  def scale1_body(degp_ref, h1_ref, g_ref, dinv_ref):
    dinv = lax.rsqrt(degp_ref[0] + degp_ref[1] + 1.0)
    dinv_ref[...] = dinv
    g_ref[...] = dinv * h1_ref[...]

  g1, dinv_pk = pl.pallas_call(
      scale1_body,
      out_shape=(jax.ShapeDtypeStruct((npk, 128), jnp.float32),
                 jax.ShapeDtypeStruct((npk, 128), jnp.float32)),
  )(deg_p, h1)

  acc1, _ = sc_agg(g1, src, dst, zeros)

  def stage2_body(accp_ref, g1_ref, dinv_ref, b1_ref, w2bd_ref, o_ref):
    dinv = dinv_ref[...]
    out1 = dinv * (accp_ref[0] + accp_ref[1] + g1_ref[...]) + b1_ref[...]
    r = jnp.maximum(out1, 0.0)
    h2 = jnp.dot(r, w2bd_ref[...], preferred_element_type=jnp.float32)
    o_ref[...] = dinv * h2

  g2 = pl.pallas_call(
      stage2_body,
      out_shape=jax.ShapeDtypeStruct((npk, 128), jnp.float32),
  )(acc1, g1, dinv_pk, b1t, W2bd)

  acc2, _ = sc_agg(g2, src, dst, zeros)

  def stage3_body(accp_ref, g2_ref, dinv_ref, b2_ref, msum_ref, o_ref):
    z = dinv_ref[...] * (accp_ref[0] + accp_ref[1] + g2_ref[...]) + b2_ref[...]
    lane = lax.broadcasted_iota(jnp.int32, (npk, 128), 1)
    valid = lane % feat < ncls
    m = jnp.max(jnp.where(valid, z, -jnp.inf), axis=1, keepdims=True)
    ex = jnp.exp(z - m)
    sums = jnp.dot(ex, msum_ref[...], preferred_element_type=jnp.float32)
    res = z - m - jnp.log(sums)
    for p in range(8):
      o_ref[p::8, :] = res[:n // 8, p * feat:p * feat + ncls]

  out = pl.pallas_call(
      stage3_body,
      out_shape=jax.ShapeDtypeStruct((n, ncls), jnp.float32),
  )(acc2, g2, dinv_pk, b2t, Msum)

  return out


# speedup vs baseline: 1.1944x; 1.1944x over previous
import functools

import jax
import jax.numpy as jnp
from jax import lax
from jax.experimental import pallas as pl
from jax.experimental.pallas import tpu as pltpu
from jax.experimental.pallas import tpu_sc as plsc

NC = 2
NS = 16
NW = NC * NS
CHUNK = 128
NBUF = 10


def _unpack_rows(pk_ref, flat_ref, nrow):
  @pl.loop(0, nrow)
  def _(i):
    for p in range(8):
      flat_ref[8 * i + p] = pk_ref[i, pl.ds(p * 16, 16)]


def _pack_rows(flat_ref, pk_ref, nrow):
  @pl.loop(0, nrow)
  def _(i):
    for p in range(8):
      pk_ref[i, pl.ds(p * 16, 16)] = flat_ref[8 * i + p]


def _make_sc_agg(n_pad, k, feat):
  stripe = n_pad // NS
  stripe_pk = stripe // 8
  mesh = plsc.VectorSubcoreMesh(core_axis_name="c", subcore_axis_name="s")
  assert k % NBUF == 0 and stripe % 8 == 0

  @functools.partial(
      pl.kernel,
      out_type=(jax.ShapeDtypeStruct((NC, n_pad // 8, 128), jnp.float32),
                jax.ShapeDtypeStruct((NC, n_pad, feat), jnp.float32)),
      mesh=mesh,
      compiler_params=pltpu.CompilerParams(use_tc_tiling_on_sc=False),
      scratch_types=[
          pltpu.VMEM((k, CHUNK), jnp.int32),
          pltpu.VMEM((k, CHUNK), jnp.int32),
          pltpu.VMEM((NBUF, CHUNK, feat), jnp.float32),
          pltpu.VMEM((stripe, feat), jnp.float32),
          pltpu.VMEM((stripe_pk, 128), jnp.float32),
          pltpu.VMEM_SHARED((n_pad, feat), jnp.float32),
          pltpu.SemaphoreType.DMA((NBUF,)),
          pltpu.SemaphoreType.DMA((NBUF,)),
          pltpu.SemaphoreType.DMA,
      ],
  )
  def agg(g_hbm, src_hbm, dst_hbm, zeros_hbm, out_hbm, gtab_hbm,
          src_v, dst_v, rows_v, flat_v, pk_v, acc, gsem, ssem, sem):
    c = lax.axis_index("c")
    s = lax.axis_index("s")
    w = c * NS + s
    pltpu.sync_copy(src_hbm.at[pl.ds(w * k, k)], src_v)
    pltpu.sync_copy(dst_hbm.at[pl.ds(w * k, k)], dst_v)
    pltpu.sync_copy(g_hbm.at[pl.ds(s * stripe_pk, stripe_pk)], pk_v)
    _unpack_rows(pk_v, flat_v, stripe_pk)
    pltpu.async_copy(flat_v, gtab_hbm.at[c, pl.ds(s * stripe, stripe)], sem)
    pltpu.sync_copy(zeros_hbm.at[pl.ds(s * stripe, stripe)],
                    acc.at[pl.ds(s * stripe, stripe)])
    pltpu.make_async_copy(
        flat_v, gtab_hbm.at[c, pl.ds(s * stripe, stripe)], sem).wait()
    plsc.subcore_barrier()

    gtab = gtab_hbm.at[c]
    for b in range(NBUF):
      pltpu.async_copy(gtab.at[src_v.at[b]], rows_v.at[b], gsem.at[b])

    @pl.loop(0, k, step=NBUF)
    def _(j):
      for b in range(NBUF):
        cj = j + b
        pltpu.make_async_copy(gtab.at[src_v.at[cj]], rows_v.at[b],
                              gsem.at[b]).wait()
        pltpu.async_copy(rows_v.at[b], acc.at[dst_v.at[cj]], ssem.at[b],
                         add=True)
        bq = (b - 1) % NBUF
        pv = cj - 1

        @pl.when(jnp.logical_and(pv >= 0, pv + NBUF < k))
        def _():
          pltpu.make_async_copy(rows_v.at[bq], acc.at[dst_v.at[0]],
                                ssem.at[bq]).wait()
          pltpu.async_copy(gtab.at[src_v.at[pv + NBUF]], rows_v.at[bq],
                           gsem.at[bq])

    for b in range(NBUF):
      pltpu.make_async_copy(rows_v.at[b], acc.at[dst_v.at[0]],
                            ssem.at[b]).wait()

    plsc.subcore_barrier()
    pltpu.sync_copy(acc.at[pl.ds(s * stripe, stripe)], flat_v)
    _pack_rows(flat_v, pk_v, stripe_pk)
    pltpu.sync_copy(pk_v, out_hbm.at[c, pl.ds(s * stripe_pk, stripe_pk)])

  return agg


def _make_sc_deg(n_pad, k, feat):
  stripe = n_pad // NS
  stripe_pk = stripe // 8
  mesh = plsc.VectorSubcoreMesh(core_axis_name="c", subcore_axis_name="s")

  @functools.partial(
      pl.kernel,
      out_type=jax.ShapeDtypeStruct((NC, n_pad // 8, 128), jnp.float32),
      mesh=mesh,
      compiler_params=pltpu.CompilerParams(use_tc_tiling_on_sc=False),
      scratch_types=[
          pltpu.VMEM((k, CHUNK), jnp.int32),
          pltpu.VMEM((CHUNK, feat), jnp.float32),
          pltpu.VMEM((stripe, feat), jnp.float32),
          pltpu.VMEM((stripe_pk, 128), jnp.float32),
          pltpu.VMEM_SHARED((n_pad, feat), jnp.float32),
          pltpu.SemaphoreType.DMA((NBUF,)),
      ],
  )
  def deg(ones_hbm, dst_hbm, zeros_hbm, out_hbm,
          dst_v, ones_v, flat_v, pk_v, acc, ssem):
    c = lax.axis_index("c")
    s = lax.axis_index("s")
    w = c * NS + s
    pltpu.sync_copy(dst_hbm.at[pl.ds(w * k, k)], dst_v)
    pltpu.sync_copy(ones_hbm, ones_v)
    pltpu.sync_copy(zeros_hbm.at[pl.ds(s * stripe, stripe)],
                    acc.at[pl.ds(s * stripe, stripe)])
    plsc.subcore_barrier()

    for b in range(NBUF):
      pltpu.async_copy(ones_v, acc.at[dst_v.at[b]], ssem.at[b], add=True)

    @pl.loop(NBUF, k, step=NBUF)
    def _(j):
      for b in range(NBUF):
        pltpu.make_async_copy(ones_v, acc.at[dst_v.at[0]], ssem.at[b]).wait()
        pltpu.async_copy(ones_v, acc.at[dst_v.at[j + b]], ssem.at[b],
                         add=True)

    for b in range(NBUF):
      pltpu.make_async_copy(ones_v, acc.at[dst_v.at[0]], ssem.at[b]).wait()

    plsc.subcore_barrier()
    pltpu.sync_copy(acc.at[pl.ds(s * stripe, stripe)], flat_v)
    _pack_rows(flat_v, pk_v, stripe_pk)
    pltpu.sync_copy(pk_v, out_hbm.at[c, pl.ds(s * stripe_pk, stripe_pk)])

  return deg


def kernel(x, edge_index, W1, b1, W2, b2):
  n, d = x.shape
  h = W1.shape[1]
  ncls = W2.shape[1]
  e = edge_index.shape[1]
  feat = 16

  k = -(-e // (NW * CHUNK))
  k += k % 2
  e_pad = NW * k * CHUNK
  stripe = -(-n // NS)
  stripe += (-stripe) % 8
  n_pad = NS * stripe
  npk = n_pad // 8
  n_trash = n_pad - n
  pad_cnt = e_pad - e
  ew = e // 128

  def prep_body(ei_ref, src_ref, dst_ref):
    src_ref[:ew] = ei_ref[0:1, :].reshape(ew, 128)
    dst_ref[:ew] = ei_ref[1:2, :].reshape(ew, 128)
    if pad_cnt:
      i = (lax.broadcasted_iota(jnp.int32, (pad_cnt // 128, 128), 0) * 128
           + lax.broadcasted_iota(jnp.int32, (pad_cnt // 128, 128), 1))
      src_ref[ew:] = (i * 97) % n
      dst_ref[ew:] = n + i % n_trash

  src, dst = pl.pallas_call(
      prep_body,
      out_shape=(jax.ShapeDtypeStruct((NW * k, CHUNK), jnp.int32),
                 jax.ShapeDtypeStruct((NW * k, CHUNK), jnp.int32)),
  )(edge_index)

  zeros = jnp.zeros((n_pad, feat), jnp.float32)
  ones_rows = jnp.ones((CHUNK, feat), jnp.float32)
  W2p = jnp.pad(W2, ((0, 0), (0, feat - ncls)))
  W2bd = jnp.kron(jnp.eye(8, dtype=jnp.float32), W2p)
  vrow = (jnp.arange(feat) < ncls).astype(jnp.float32)[:, None]
  Msum = jnp.kron(jnp.eye(8, dtype=jnp.float32),
                  vrow * jnp.ones((1, feat), jnp.float32))
  b1t = jnp.tile(b1, 8).reshape(1, 128)
  b2t = jnp.tile(jnp.pad(b2, (0, feat - ncls)), 8).reshape(1, 128)

  sc_agg = _make_sc_agg(n_pad, k, feat)
  sc_deg = _make_sc_deg(n_pad, k, feat)

  def mm1_body(x_ref, w_ref, o_ref):
    zpad = jnp.zeros((npk - n // 8, feat), jnp.float32)
    for p in range(8):
      hp = jnp.dot(x_ref[p::8, :], w_ref[...],
                   preferred_element_type=jnp.float32)
      o_ref[:, p * feat:(p + 1) * feat] = jnp.concatenate([hp, zpad], axis=0)

  h1 = pl.pallas_call(
      mm1_body,
      out_shape=jax.ShapeDtypeStruct((npk, 128), jnp.float32),
  )(x, W1)

  deg_p = sc_deg(ones_rows, dst, zeros)

  def scale1_body(degp_ref, h1_ref, g_ref, dinv_ref):
    dinv = lax.rsqrt(degp_ref[0] + degp_ref[1] + 1.0)
    dinv_ref[...] = dinv
    g_ref[...] = dinv * h1_ref[...]

  g1, dinv_pk = pl.pallas_call(
      scale1_body,
      out_shape=(jax.ShapeDtypeStruct((npk, 128), jnp.float32),
                 jax.ShapeDtypeStruct((npk, 128), jnp.float32)),
  )(deg_p, h1)

  acc1, _ = sc_agg(g1, src, dst, zeros)

  def stage2_body(accp_ref, g1_ref, dinv_ref, b1_ref, w2bd_ref, o_ref):
    dinv = dinv_ref[...]
    out1 = dinv * (accp_ref[0] + accp_ref[1] + g1_ref[...]) + b1_ref[...]
    r = jnp.maximum(out1, 0.0)
    h2 = jnp.dot(r, w2bd_ref[...], preferred_element_type=jnp.float32)
    o_ref[...] = dinv * h2

  g2 = pl.pallas_call(
      stage2_body,
      out_shape=jax.ShapeDtypeStruct((npk, 128), jnp.float32),
  )(acc1, g1, dinv_pk, b1t, W2bd)

  acc2, _ = sc_agg(g2, src, dst, zeros)

  def stage3_body(accp_ref, g2_ref, dinv_ref, b2_ref, msum_ref, o_ref):
    z = dinv_ref[...] * (accp_ref[0] + accp_ref[1] + g2_ref[...]) + b2_ref[...]
    lane = lax.broadcasted_iota(jnp.int32, (npk, 128), 1)
    valid = lane % feat < ncls
    m = jnp.max(jnp.where(valid, z, -jnp.inf), axis=1, keepdims=True)
    ex = jnp.exp(z - m)
    sums = jnp.dot(ex, msum_ref[...], preferred_element_type=jnp.float32)
    res = z - m - jnp.log(sums)
    for p in range(8):
      o_ref[p::8, :] = res[:n // 8, p * feat:p * feat + ncls]

  out = pl.pallas_call(
      stage3_body,
      out_shape=jax.ShapeDtypeStruct((n, ncls), jnp.float32),
  )(acc2, g2, dinv_pk, b2t, Msum)

  return out


# speedup vs baseline: 1.2015x; 1.0059x over previous
import functools

import jax
import jax.numpy as jnp
from jax import lax
from jax.experimental import pallas as pl
from jax.experimental.pallas import tpu as pltpu
from jax.experimental.pallas import tpu_sc as plsc

NC = 2
NS = 16
NW = NC * NS
CHUNK = 128
NBUF = 8


def _unpack_rows(pk_ref, flat_ref, nrow):
  @pl.loop(0, nrow)
  def _(i):
    for p in range(8):
      flat_ref[8 * i + p] = pk_ref[i, pl.ds(p * 16, 16)]


def _pack_rows(flat_ref, pk_ref, nrow):
  @pl.loop(0, nrow)
  def _(i):
    for p in range(8):
      pk_ref[i, pl.ds(p * 16, 16)] = flat_ref[8 * i + p]


def _make_sc_agg(n_pad, k, feat):
  stripe = n_pad // NS
  stripe_pk = stripe // 8
  mesh = plsc.VectorSubcoreMesh(core_axis_name="c", subcore_axis_name="s")
  assert k % NBUF == 0 and stripe % 8 == 0

  @functools.partial(
      pl.kernel,
      out_type=(jax.ShapeDtypeStruct((NC, n_pad // 8, 128), jnp.float32),
                jax.ShapeDtypeStruct((NC, n_pad, feat), jnp.float32)),
      mesh=mesh,
      compiler_params=pltpu.CompilerParams(use_tc_tiling_on_sc=False),
      scratch_types=[
          pltpu.VMEM((k, CHUNK), jnp.int32),
          pltpu.VMEM((k, CHUNK), jnp.int32),
          pltpu.VMEM((NBUF, CHUNK, feat), jnp.float32),
          pltpu.VMEM((stripe, feat), jnp.float32),
          pltpu.VMEM((stripe_pk, 128), jnp.float32),
          pltpu.VMEM_SHARED((n_pad, feat), jnp.float32),
          pltpu.SemaphoreType.DMA((NBUF,)),
          pltpu.SemaphoreType.DMA((NBUF,)),
          pltpu.SemaphoreType.DMA,
      ],
  )
  def agg(g_hbm, src_hbm, dst_hbm, zeros_hbm, out_hbm, gtab_hbm,
          src_v, dst_v, rows_v, flat_v, pk_v, acc, gsem, ssem, sem):
    c = lax.axis_index("c")
    s = lax.axis_index("s")
    w = c * NS + s
    pltpu.sync_copy(src_hbm.at[pl.ds(w * k, k)], src_v)
    pltpu.sync_copy(dst_hbm.at[pl.ds(w * k, k)], dst_v)
    pltpu.sync_copy(g_hbm.at[pl.ds(s * stripe_pk, stripe_pk)], pk_v)
    _unpack_rows(pk_v, flat_v, stripe_pk)
    pltpu.async_copy(flat_v, gtab_hbm.at[c, pl.ds(s * stripe, stripe)], sem)
    pltpu.sync_copy(zeros_hbm.at[pl.ds(s * stripe, stripe)],
                    acc.at[pl.ds(s * stripe, stripe)])
    pltpu.make_async_copy(
        flat_v, gtab_hbm.at[c, pl.ds(s * stripe, stripe)], sem).wait()
    plsc.subcore_barrier()

    gtab = gtab_hbm.at[c]
    for b in range(NBUF):
      pltpu.async_copy(gtab.at[src_v.at[b]], rows_v.at[b], gsem.at[b])

    @pl.loop(0, k, step=NBUF)
    def _(j):
      for b in range(NBUF):
        cj = j + b
        pltpu.make_async_copy(gtab.at[src_v.at[cj]], rows_v.at[b],
                              gsem.at[b]).wait()
        pltpu.async_copy(rows_v.at[b], acc.at[dst_v.at[cj]], ssem.at[b],
                         add=True)
        bq = (b - 1) % NBUF
        pv = cj - 1

        @pl.when(jnp.logical_and(pv >= 0, pv + NBUF < k))
        def _():
          pltpu.make_async_copy(rows_v.at[bq], acc.at[dst_v.at[0]],
                                ssem.at[bq]).wait()
          pltpu.async_copy(gtab.at[src_v.at[pv + NBUF]], rows_v.at[bq],
                           gsem.at[bq])

    for b in range(NBUF):
      pltpu.make_async_copy(rows_v.at[b], acc.at[dst_v.at[0]],
                            ssem.at[b]).wait()

    plsc.subcore_barrier()
    pltpu.sync_copy(acc.at[pl.ds(s * stripe, stripe)], flat_v)
    _pack_rows(flat_v, pk_v, stripe_pk)
    pltpu.sync_copy(pk_v, out_hbm.at[c, pl.ds(s * stripe_pk, stripe_pk)])

  return agg


def _make_sc_deg(n_pad, k, feat):
  stripe = n_pad // NS
  stripe_pk = stripe // 8
  mesh = plsc.VectorSubcoreMesh(core_axis_name="c", subcore_axis_name="s")

  @functools.partial(
      pl.kernel,
      out_type=jax.ShapeDtypeStruct((NC, n_pad // 8, 128), jnp.float32),
      mesh=mesh,
      compiler_params=pltpu.CompilerParams(use_tc_tiling_on_sc=False),
      scratch_types=[
          pltpu.VMEM((k, CHUNK), jnp.int32),
          pltpu.VMEM((CHUNK, feat), jnp.float32),
          pltpu.VMEM((stripe, feat), jnp.float32),
          pltpu.VMEM((stripe_pk, 128), jnp.float32),
          pltpu.VMEM_SHARED((n_pad, feat), jnp.float32),
          pltpu.SemaphoreType.DMA((NBUF,)),
      ],
  )
  def deg(ones_hbm, dst_hbm, zeros_hbm, out_hbm,
          dst_v, ones_v, flat_v, pk_v, acc, ssem):
    c = lax.axis_index("c")
    s = lax.axis_index("s")
    w = c * NS + s
    pltpu.sync_copy(dst_hbm.at[pl.ds(w * k, k)], dst_v)
    pltpu.sync_copy(ones_hbm, ones_v)
    pltpu.sync_copy(zeros_hbm.at[pl.ds(s * stripe, stripe)],
                    acc.at[pl.ds(s * stripe, stripe)])
    plsc.subcore_barrier()

    for b in range(NBUF):
      pltpu.async_copy(ones_v, acc.at[dst_v.at[b]], ssem.at[b], add=True)

    @pl.loop(NBUF, k, step=NBUF)
    def _(j):
      for b in range(NBUF):
        pltpu.make_async_copy(ones_v, acc.at[dst_v.at[0]], ssem.at[b]).wait()
        pltpu.async_copy(ones_v, acc.at[dst_v.at[j + b]], ssem.at[b],
                         add=True)

    for b in range(NBUF):
      pltpu.make_async_copy(ones_v, acc.at[dst_v.at[0]], ssem.at[b]).wait()

    plsc.subcore_barrier()
    pltpu.sync_copy(acc.at[pl.ds(s * stripe, stripe)], flat_v)
    _pack_rows(flat_v, pk_v, stripe_pk)
    pltpu.sync_copy(pk_v, out_hbm.at[c, pl.ds(s * stripe_pk, stripe_pk)])

  return deg


def kernel(x, edge_index, W1, b1, W2, b2):
  n, d = x.shape
  h = W1.shape[1]
  ncls = W2.shape[1]
  e = edge_index.shape[1]
  feat = 16

  k = -(-e // (NW * CHUNK))
  k += k % 2
  e_pad = NW * k * CHUNK
  stripe = -(-n // NS)
  stripe += (-stripe) % 8
  n_pad = NS * stripe
  npk = n_pad // 8
  n_trash = n_pad - n
  pad_cnt = e_pad - e
  ew = e // 128

  def prep_body(ei_ref, src_ref, dst_ref):
    src_ref[:ew] = ei_ref[0:1, :].reshape(ew, 128)
    dst_ref[:ew] = ei_ref[1:2, :].reshape(ew, 128)
    if pad_cnt:
      i = (lax.broadcasted_iota(jnp.int32, (pad_cnt // 128, 128), 0) * 128
           + lax.broadcasted_iota(jnp.int32, (pad_cnt // 128, 128), 1))
      src_ref[ew:] = (i * 97) % n
      dst_ref[ew:] = n + i % n_trash

  src, dst = pl.pallas_call(
      prep_body,
      out_shape=(jax.ShapeDtypeStruct((NW * k, CHUNK), jnp.int32),
                 jax.ShapeDtypeStruct((NW * k, CHUNK), jnp.int32)),
  )(edge_index)

  zeros = jnp.zeros((n_pad, feat), jnp.float32)
  ones_rows = jnp.ones((CHUNK, feat), jnp.float32)
  W2p = jnp.pad(W2, ((0, 0), (0, feat - ncls)))
  W2bd = jnp.kron(jnp.eye(8, dtype=jnp.float32), W2p)
  vrow = (jnp.arange(feat) < ncls).astype(jnp.float32)[:, None]
  Msum = jnp.kron(jnp.eye(8, dtype=jnp.float32),
                  vrow * jnp.ones((1, feat), jnp.float32))
  b1t = jnp.tile(b1, 8).reshape(1, 128)
  b2t = jnp.tile(jnp.pad(b2, (0, feat - ncls)), 8).reshape(1, 128)

  sc_agg = _make_sc_agg(n_pad, k, feat)
  sc_deg = _make_sc_deg(n_pad, k, feat)

  def mm1_body(x_ref, w_ref, o_ref):
    zpad = jnp.zeros((npk - n // 8, feat), jnp.float32)
    for p in range(8):
      hp = jnp.dot(x_ref[p::8, :], w_ref[...],
                   preferred_element_type=jnp.float32)
      o_ref[:, p * feat:(p + 1) * feat] = jnp.concatenate([hp, zpad], axis=0)

  h1 = pl.pallas_call(
      mm1_body,
      out_shape=jax.ShapeDtypeStruct((npk, 128), jnp.float32),
  )(x, W1)

  deg_p = sc_deg(ones_rows, dst, zeros)

  def scale1_body(degp_ref, h1_ref, g_ref, dinv_ref):
    dinv = lax.rsqrt(degp_ref[0] + degp_ref[1] + 1.0)
    dinv_ref[...] = dinv
    g_ref[...] = dinv * h1_ref[...]

  g1, dinv_pk = pl.pallas_call(
      scale1_body,
      out_shape=(jax.ShapeDtypeStruct((npk, 128), jnp.float32),
                 jax.ShapeDtypeStruct((npk, 128), jnp.float32)),
  )(deg_p, h1)

  acc1, _ = sc_agg(g1, src, dst, zeros)

  def stage2_body(accp_ref, g1_ref, dinv_ref, b1_ref, w2bd_ref, o_ref):
    dinv = dinv_ref[...]
    out1 = dinv * (accp_ref[0] + accp_ref[1] + g1_ref[...]) + b1_ref[...]
    r = jnp.maximum(out1, 0.0)
    h2 = jnp.dot(r, w2bd_ref[...], preferred_element_type=jnp.float32)
    o_ref[...] = dinv * h2

  g2 = pl.pallas_call(
      stage2_body,
      out_shape=jax.ShapeDtypeStruct((npk, 128), jnp.float32),
  )(acc1, g1, dinv_pk, b1t, W2bd)

  acc2, _ = sc_agg(g2, src, dst, zeros)

  def stage3_body(accp_ref, g2_ref, dinv_ref, b2_ref, msum_ref, o_ref):
    z = dinv_ref[...] * (accp_ref[0] + accp_ref[1] + g2_ref[...]) + b2_ref[...]
    lane = lax.broadcasted_iota(jnp.int32, (npk, 128), 1)
    valid = lane % feat < ncls
    m = jnp.max(jnp.where(valid, z, -jnp.inf), axis=1, keepdims=True)
    ex = jnp.exp(z - m)
    sums = jnp.dot(ex, msum_ref[...], preferred_element_type=jnp.float32)
    res = z - m - jnp.log(sums)
    for p in range(8):
      o_ref[p::8, :] = res[:n // 8, p * feat:p * feat + ncls]

  out = pl.pallas_call(
      stage3_body,
      out_shape=jax.ShapeDtypeStruct((n, ncls), jnp.float32),
  )(acc2, g2, dinv_pk, b2t, Msum)

  return out
